# baseline (speedup 1.0000x reference)
import functools

import jax
import jax.numpy as jnp
from jax.experimental import pallas as pl
from jax.experimental.pallas import tpu as pltpu


def _se_fused_kernel(x_ref, w1t_ref, b1_ref, w2t_ref, b2_ref, o_ref, *, inv_s):
    x = x_ref[...]
    pooled = jnp.sum(x, axis=-1) * inv_s
    h = jnp.maximum(
        jnp.dot(pooled, w1t_ref[...], preferred_element_type=jnp.float32)
        + b1_ref[...],
        0.0,
    )
    g = jax.nn.sigmoid(
        jnp.dot(h, w2t_ref[...], preferred_element_type=jnp.float32)
        + b2_ref[...]
    )
    o_ref[...] = x * g[:, :, None].astype(x.dtype)


def kernel(x_nchw, w1, b1, w2, b2):
    B, C, H, W = x_nchw.shape
    S = H * W
    Ch = w1.shape[0]

    nb = 8
    while B % nb:
        nb //= 2

    S_pad = (S + 127) // 128 * 128
    x3 = jnp.pad(x_nchw.reshape(B, C, S), ((0, 0), (0, 0), (0, S_pad - S)))
    w1t = w1.astype(jnp.float32).T
    w2t = w2.astype(jnp.float32).T
    b1r = b1.astype(jnp.float32).reshape(1, Ch)
    b2r = b2.astype(jnp.float32).reshape(1, C)

    blk_bytes = nb * C * S_pad * 4

    out = pl.pallas_call(
        functools.partial(_se_fused_kernel, inv_s=1.0 / S),
        out_shape=jax.ShapeDtypeStruct((B, C, S_pad), x3.dtype),
        grid=(B // nb,),
        in_specs=[
            pl.BlockSpec((nb, C, S_pad), lambda b: (b, 0, 0)),
            pl.BlockSpec((C, Ch), lambda b: (0, 0)),
            pl.BlockSpec((1, Ch), lambda b: (0, 0)),
            pl.BlockSpec((Ch, C), lambda b: (0, 0)),
            pl.BlockSpec((1, C), lambda b: (0, 0)),
        ],
        out_specs=pl.BlockSpec((nb, C, S_pad), lambda b: (b, 0, 0)),
        compiler_params=pltpu.CompilerParams(
            dimension_semantics=("parallel",),
            vmem_limit_bytes=int(min(5 * blk_bytes + (4 << 20), 120 << 20)),
        ),
    )(x3, w1t, b1r, w2t, b2r)
    return out[:, :, :S].reshape(B, C, H, W)


# speedup vs baseline: 1.1855x; 1.1855x over previous
import functools

import jax
import jax.numpy as jnp
from jax.experimental import pallas as pl
from jax.experimental.pallas import tpu as pltpu


def _se_fused_kernel(x_ref, w1t_ref, b1_ref, w2t_ref, b2_ref, o_ref, *, inv_s):
    x = x_ref[...]
    pooled = jnp.sum(x, axis=-1) * inv_s
    h = jnp.maximum(
        jnp.dot(pooled, w1t_ref[...], preferred_element_type=jnp.float32)
        + b1_ref[...],
        0.0,
    )
    g = jax.nn.sigmoid(
        jnp.dot(h, w2t_ref[...], preferred_element_type=jnp.float32)
        + b2_ref[...]
    )
    o_ref[:, :, : x.shape[-1]] = x * g[:, :, None].astype(x.dtype)


def kernel(x_nchw, w1, b1, w2, b2):
    B, C, H, W = x_nchw.shape
    S = H * W
    Ch = w1.shape[0]

    nb = 16
    while B % nb:
        nb //= 2

    x3 = x_nchw.reshape(B, C, S)
    w1t = w1.astype(jnp.float32).T
    w2t = w2.astype(jnp.float32).T
    b1r = b1.astype(jnp.float32).reshape(1, Ch)
    b2r = b2.astype(jnp.float32).reshape(1, C)

    blk_bytes = nb * C * S * 4
    S_p = (S + 127) // 128 * 128

    out = pl.pallas_call(
        functools.partial(_se_fused_kernel, inv_s=1.0 / S),
        out_shape=jax.ShapeDtypeStruct((B, C, S_p), x3.dtype),
        grid=(B // nb,),
        in_specs=[
            pl.BlockSpec((nb, C, S), lambda b: (b, 0, 0)),
            pl.BlockSpec((C, Ch), lambda b: (0, 0)),
            pl.BlockSpec((1, Ch), lambda b: (0, 0)),
            pl.BlockSpec((Ch, C), lambda b: (0, 0)),
            pl.BlockSpec((1, C), lambda b: (0, 0)),
        ],
        out_specs=pl.BlockSpec((nb, C, S_p), lambda b: (b, 0, 0)),
        compiler_params=pltpu.CompilerParams(
            dimension_semantics=("parallel",),
            vmem_limit_bytes=int(min(5 * blk_bytes + (4 << 20), 120 << 20)),
        ),
    )(x3, w1t, b1r, w2t, b2r)
    return out[:, :, :S].reshape(B, C, H, W)
